# Initial kernel scaffold; baseline (speedup 1.0000x reference)
import jax
import jax.numpy as jnp
from jax.experimental import pallas as pl


def kernel(params, node_ids, edge_index, edge_type):
    raise NotImplementedError("write your pallas kernel here")


# speedup vs baseline: 1.1202x; 1.1202x over previous
import functools

import jax
import jax.numpy as jnp
from jax import lax
from jax.experimental import pallas as pl
from jax.experimental.pallas import tpu as pltpu

N_NODES = 10000
N_EDGES = 320000
NUM_RELATIONS = 64
NUM_BASES = 30
DIM = 128
HEADS = 8
HEAD_DIM = DIM // HEADS

NT = 2000


def _matmul_kernel(a_ref, b_ref, o_ref):
    o_ref[...] = jnp.dot(a_ref[...], b_ref[...],
                         preferred_element_type=jnp.float32)


def _small_matmul(a, b):
    m, k = a.shape
    k2, n = b.shape
    return pl.pallas_call(
        _matmul_kernel,
        out_shape=jax.ShapeDtypeStruct((m, n), jnp.float32),
    )(a, b)


def _xw_kernel(x_ref, w_ref, o_ref):
    o_ref[0] = jnp.dot(x_ref[...], w_ref[0],
                       preferred_element_type=jnp.float32)


def _xw_all_relations(x, w):
    grid = (N_NODES // NT, NUM_RELATIONS)
    return pl.pallas_call(
        _xw_kernel,
        grid=grid,
        in_specs=[
            pl.BlockSpec((NT, DIM), lambda i, r: (i, 0)),
            pl.BlockSpec((1, DIM, DIM), lambda i, r: (r, 0, 0)),
        ],
        out_specs=pl.BlockSpec((1, NT, DIM), lambda i, r: (r, i, 0)),
        out_shape=jax.ShapeDtypeStruct((NUM_RELATIONS, N_NODES, DIM),
                                       jnp.float32),
    )(x, w)


def _combine_kernel(agg_ref, x_ref, w_ref, b_ref, g_ref, bb_ref, o_ref,
                    m_ref, *, residual, mean_out):
    x = x_ref[...]
    h = agg_ref[...] + jnp.dot(x, w_ref[...],
                               preferred_element_type=jnp.float32)
    h = h + b_ref[...]
    mu = jnp.mean(h, axis=-1, keepdims=True)
    var = jnp.mean((h - mu) ** 2, axis=-1, keepdims=True)
    h = (h - mu) / jnp.sqrt(var + 1e-5) * g_ref[...] + bb_ref[...]
    h = jnp.maximum(h, 0.0)
    if residual:
        h = x + h
    o_ref[...] = h
    if mean_out:
        i = pl.program_id(0)

        @pl.when(i == 0)
        def _():
            m_ref[...] = jnp.zeros_like(m_ref)

        m_ref[...] += jnp.sum(h, axis=0, keepdims=True) * (1.0 / N_NODES)


def _combine(agg, x, w, b, g, bb, residual=False, mean_out=False):
    grid = (N_NODES // NT,)
    kern = functools.partial(_combine_kernel, residual=residual,
                             mean_out=mean_out)
    out_shape = [jax.ShapeDtypeStruct((N_NODES, DIM), jnp.float32),
                 jax.ShapeDtypeStruct((1, DIM), jnp.float32)]
    outs = pl.pallas_call(
        kern,
        grid=grid,
        in_specs=[
            pl.BlockSpec((NT, DIM), lambda i: (i, 0)),
            pl.BlockSpec((NT, DIM), lambda i: (i, 0)),
            pl.BlockSpec((DIM, DIM), lambda i: (0, 0)),
            pl.BlockSpec((DIM,), lambda i: (0,)),
            pl.BlockSpec((DIM,), lambda i: (0,)),
            pl.BlockSpec((DIM,), lambda i: (0,)),
        ],
        out_specs=[
            pl.BlockSpec((NT, DIM), lambda i: (i, 0)),
            pl.BlockSpec((1, DIM), lambda i: (0, 0)),
        ],
        out_shape=out_shape,
    )(agg, x, w, b, g, bb)
    return outs if mean_out else outs[0]


def _qkv_kernel(x_ref, w_ref, b_ref, o_ref):
    o_ref[...] = jnp.dot(x_ref[...], w_ref[...],
                         preferred_element_type=jnp.float32) + b_ref[...]


def _qkv(x, wqkv, bqkv):
    grid = (N_NODES // NT,)
    return pl.pallas_call(
        _qkv_kernel,
        grid=grid,
        in_specs=[
            pl.BlockSpec((NT, DIM), lambda i: (i, 0)),
            pl.BlockSpec((DIM, 3 * DIM), lambda i: (0, 0)),
            pl.BlockSpec((3 * DIM,), lambda i: (0,)),
        ],
        out_specs=pl.BlockSpec((NT, 3 * DIM), lambda i: (i, 0)),
        out_shape=jax.ShapeDtypeStruct((N_NODES, 3 * DIM), jnp.float32),
    )(x, wqkv, bqkv)


def _entity_gather(table, node_ids):
    return jnp.take(table, node_ids, axis=0)


def _rgcn_aggregate(xw, src, dst, edge_type):
    idx = dst * NUM_RELATIONS + edge_type
    cnt = jax.ops.segment_sum(jnp.ones((N_EDGES,), jnp.float32), idx,
                              num_segments=N_NODES * NUM_RELATIONS)
    w = 1.0 / jnp.maximum(cnt, 1.0)
    rows = xw[edge_type, src] * w[idx][:, None]
    return jax.ops.segment_sum(rows, dst, num_segments=N_NODES)


def _tf_aggregate(q, k, v, er, src, dst, edge_type):
    qd = q[dst].reshape(-1, HEADS, HEAD_DIM)
    e = er[edge_type]
    ke = (k[src] + e).reshape(-1, HEADS, HEAD_DIM)
    ve = (v[src] + e).reshape(-1, HEADS, HEAD_DIM)
    alpha = jnp.sum(qd * ke, axis=-1) * (1.0 / jnp.sqrt(float(HEAD_DIM)))
    ex = jnp.exp(alpha)
    den = jax.ops.segment_sum(ex, dst, num_segments=N_NODES)
    attn = ex / (den[dst] + 1e-16)
    return jax.ops.segment_sum(ve * attn[:, :, None], dst,
                               num_segments=N_NODES).reshape(N_NODES, DIM)


def kernel(params, node_ids, edge_index, edge_type):
    src, dst = edge_index[0], edge_index[1]

    x = _entity_gather(params['entity_table'], node_ids)

    for l in range(2):
        p = params['rgcn'][l]
        w = _small_matmul(p['comp'],
                          p['bases'].reshape(NUM_BASES, DIM * DIM))
        w = w.reshape(NUM_RELATIONS, DIM, DIM)
        xw = _xw_all_relations(x, w)
        agg = _rgcn_aggregate(xw, src, dst, edge_type)
        x = _combine(agg, x, p['root'], p['bias'], p['ln_g'], p['ln_b'])

    mean = None
    for l in range(2):
        p = params['tf'][l]
        er = _small_matmul(params['relation_table'], p['We'])
        wqkv = jnp.concatenate([p['Wq'], p['Wk'], p['Wv']], axis=1)
        bqkv = jnp.concatenate([p['bq'], p['bk'], p['bv']], axis=0)
        qkv = _qkv(x, wqkv, bqkv)
        q, k, v = qkv[:, :DIM], qkv[:, DIM:2 * DIM], qkv[:, 2 * DIM:]
        agg = _tf_aggregate(q, k, v, er, src, dst, edge_type)
        last = (l == 1)
        res = _combine(agg, x, p['Wskip'], p['bskip'], p['ln_g'], p['ln_b'],
                       residual=True, mean_out=last)
        x = res[0] if last else res
    graph_embedding = res[1]
    return x, graph_embedding


# speedup vs baseline: 1.1273x; 1.0063x over previous
import functools

import jax
import jax.numpy as jnp
from jax import lax
from jax.experimental import pallas as pl
from jax.experimental.pallas import tpu as pltpu
from jax.experimental.pallas import tpu_sc as plsc

_SC = plsc.get_sparse_core_info()
SC_NC, SC_NS, SC_L = _SC.num_cores, _SC.num_subcores, _SC.num_lanes
NW = SC_NC * SC_NS

N_NODES = 10000
N_EDGES = 320000
NUM_RELATIONS = 64
NUM_BASES = 30
DIM = 128
HEADS = 8
HEAD_DIM = DIM // HEADS

NT = 2000


def _matmul_kernel(a_ref, b_ref, o_ref):
    o_ref[...] = jnp.dot(a_ref[...], b_ref[...],
                         preferred_element_type=jnp.float32)


def _small_matmul(a, b):
    m, k = a.shape
    k2, n = b.shape
    return pl.pallas_call(
        _matmul_kernel,
        out_shape=jax.ShapeDtypeStruct((m, n), jnp.float32),
    )(a, b)


def _xw_kernel(x_ref, w_ref, o_ref):
    o_ref[0] = jnp.dot(x_ref[...], w_ref[0],
                       preferred_element_type=jnp.float32)


def _xw_all_relations(x, w):
    grid = (N_NODES // NT, NUM_RELATIONS)
    return pl.pallas_call(
        _xw_kernel,
        grid=grid,
        in_specs=[
            pl.BlockSpec((NT, DIM), lambda i, r: (i, 0)),
            pl.BlockSpec((1, DIM, DIM), lambda i, r: (r, 0, 0)),
        ],
        out_specs=pl.BlockSpec((1, NT, DIM), lambda i, r: (r, i, 0)),
        out_shape=jax.ShapeDtypeStruct((NUM_RELATIONS, N_NODES, DIM),
                                       jnp.float32),
    )(x, w)


def _combine_kernel(agg_ref, x_ref, w_ref, b_ref, g_ref, bb_ref, o_ref,
                    m_ref, *, residual, mean_out):
    x = x_ref[...]
    h = jnp.sum(agg_ref[...], axis=0) + jnp.dot(
        x, w_ref[...], preferred_element_type=jnp.float32)
    h = h + b_ref[...]
    mu = jnp.mean(h, axis=-1, keepdims=True)
    var = jnp.mean((h - mu) ** 2, axis=-1, keepdims=True)
    h = (h - mu) / jnp.sqrt(var + 1e-5) * g_ref[...] + bb_ref[...]
    h = jnp.maximum(h, 0.0)
    if residual:
        h = x + h
    o_ref[...] = h
    if mean_out:
        i = pl.program_id(0)

        @pl.when(i == 0)
        def _():
            m_ref[...] = jnp.zeros_like(m_ref)

        m_ref[...] += jnp.sum(h, axis=0, keepdims=True) * (1.0 / N_NODES)


def _combine(agg, x, w, b, g, bb, residual=False, mean_out=False):
    nparts = agg.shape[0]
    grid = (N_NODES // NT,)
    kern = functools.partial(_combine_kernel, residual=residual,
                             mean_out=mean_out)
    out_shape = [jax.ShapeDtypeStruct((N_NODES, DIM), jnp.float32),
                 jax.ShapeDtypeStruct((1, DIM), jnp.float32)]
    outs = pl.pallas_call(
        kern,
        grid=grid,
        in_specs=[
            pl.BlockSpec((nparts, NT, DIM), lambda i: (0, i, 0)),
            pl.BlockSpec((NT, DIM), lambda i: (i, 0)),
            pl.BlockSpec((DIM, DIM), lambda i: (0, 0)),
            pl.BlockSpec((DIM,), lambda i: (0,)),
            pl.BlockSpec((DIM,), lambda i: (0,)),
            pl.BlockSpec((DIM,), lambda i: (0,)),
        ],
        out_specs=[
            pl.BlockSpec((NT, DIM), lambda i: (i, 0)),
            pl.BlockSpec((1, DIM), lambda i: (0, 0)),
        ],
        out_shape=out_shape,
    )(agg, x, w, b, g, bb)
    return outs if mean_out else outs[0]


def _qkv_kernel(x_ref, w_ref, b_ref, o_ref):
    o_ref[...] = jnp.dot(x_ref[...], w_ref[...],
                         preferred_element_type=jnp.float32) + b_ref[...]


def _qkv(x, wqkv, bqkv):
    grid = (N_NODES // NT,)
    return pl.pallas_call(
        _qkv_kernel,
        grid=grid,
        in_specs=[
            pl.BlockSpec((NT, DIM), lambda i: (i, 0)),
            pl.BlockSpec((DIM, 3 * DIM), lambda i: (0, 0)),
            pl.BlockSpec((3 * DIM,), lambda i: (0,)),
        ],
        out_specs=pl.BlockSpec((NT, 3 * DIM), lambda i: (i, 0)),
        out_shape=jax.ShapeDtypeStruct((N_NODES, 3 * DIM), jnp.float32),
    )(x, wqkv, bqkv)


_GB = 10240
_G_PER_W = _GB // NW
_GC = 64


def _entity_gather(table, node_ids):
    ids = jnp.concatenate(
        [node_ids, jnp.zeros((_GB - N_NODES,), jnp.int32)])
    mesh = plsc.VectorSubcoreMesh(core_axis_name="c", subcore_axis_name="s")

    @functools.partial(
        pl.kernel, mesh=mesh,
        out_type=jax.ShapeDtypeStruct((_GB, DIM), jnp.float32),
        scratch_types=[
            pltpu.VMEM((_G_PER_W // _GC, _GC), jnp.int32),
            pltpu.VMEM((_GC, DIM), jnp.float32),
            pltpu.SemaphoreType.DMA,
        ],
    )
    def k(table_hbm, idx_hbm, out_hbm, idx_v, rows_v, sem):
        wid = lax.axis_index("s") * SC_NC + lax.axis_index("c")
        base = wid * _G_PER_W
        for j in range(_G_PER_W // _GC):
            pltpu.sync_copy(idx_hbm.at[pl.ds(base + j * _GC, _GC)],
                            idx_v.at[j])
            pltpu.async_copy(table_hbm.at[idx_v.at[j]], rows_v, sem).wait()
            pltpu.sync_copy(rows_v, out_hbm.at[pl.ds(base + j * _GC, _GC)])

    return k(table, ids)[:N_NODES]


_EPW = N_EDGES // NW
_EC = 80
_ECH = _EPW // _EC
_NSEG = N_NODES * NUM_RELATIONS
_SEG_PER_T = _NSEG // (NW // 2)
_ROW_PER_T = N_NODES // (NW // 2)


def _sc_mesh():
    return plsc.VectorSubcoreMesh(core_axis_name="c", subcore_axis_name="s")


def _iota16():
    return lax.iota(jnp.int32, SC_L)


def _rel_count(src, dst, edge_type):
    @functools.partial(
        pl.kernel, mesh=_sc_mesh(),
        out_type=jax.ShapeDtypeStruct((2 * _NSEG,), jnp.float32),
        scratch_types=[
            pltpu.VMEM_SHARED((_NSEG,), jnp.float32),
            pltpu.VMEM((_EPW,), jnp.int32),
            pltpu.VMEM((_EPW,), jnp.int32),
            pltpu.VMEM((_ECH, _EC), jnp.int32),
            pltpu.VMEM((_EC,), jnp.float32),
            pltpu.VMEM((2000,), jnp.float32),
        ],
    )
    def k(dst_hbm, type_hbm, out_hbm, cnt_sp, dst_v, type_v, idx2, ones_v,
          zbuf):
        c = lax.axis_index("c")
        s = lax.axis_index("s")
        wid = s * SC_NC + c
        ebase = wid * _EPW

        def _zfill(i, _):
            zbuf[pl.ds(i * SC_L, SC_L)] = jnp.zeros((SC_L,), jnp.float32)
            return 0
        lax.fori_loop(0, 2000 // SC_L, _zfill, 0)
        for t in range(_SEG_PER_T // 2000):
            pltpu.sync_copy(zbuf, cnt_sp.at[pl.ds(s * _SEG_PER_T + t * 2000,
                                                  2000)])

        pltpu.sync_copy(dst_hbm.at[pl.ds(ebase, _EPW)], dst_v)
        pltpu.sync_copy(type_hbm.at[pl.ds(ebase, _EPW)], type_v)
        for i in range(_EC // SC_L):
            ones_v[pl.ds(i * SC_L, SC_L)] = jnp.ones((SC_L,), jnp.float32)

        def _mkidx(j, _):
            for i in range(_EC // SC_L):
                sl = pl.ds(j * _EC + i * SC_L, SC_L)
                idx2[j, pl.ds(i * SC_L, SC_L)] = (
                    dst_v[sl] * NUM_RELATIONS + type_v[sl])
            return 0
        lax.fori_loop(0, _ECH, _mkidx, 0)

        plsc.subcore_barrier()

        def _scat(j, _):
            pltpu.sync_copy(ones_v, cnt_sp.at[idx2.at[j]], add=True)
            return 0
        lax.fori_loop(0, _ECH, _scat, 0)

        plsc.subcore_barrier()
        for t in range(_SEG_PER_T // 2000):
            pltpu.sync_copy(cnt_sp.at[pl.ds(s * _SEG_PER_T + t * 2000, 2000)],
                            zbuf)
            pltpu.sync_copy(
                zbuf,
                out_hbm.at[pl.ds(c * _NSEG + s * _SEG_PER_T + t * 2000,
                                 2000)])

    return k(dst, edge_type).reshape(2, _NSEG)


_EPAD = 327680
_EPW2 = _EPAD // SC_NS
_EC2 = 64
_ECH2 = _EPW2 // _EC2
_HALF = N_NODES // 2
_ACC_ROWS = 5120


def _rgcn_aggregate(xwf, src, dst, edge_type, cnt0, cnt1):
    @functools.partial(
        pl.kernel, mesh=_sc_mesh(),
        out_type=jax.ShapeDtypeStruct((2, _ACC_ROWS, DIM), jnp.float32),
        scratch_types=[
            pltpu.VMEM_SHARED((_ACC_ROWS, DIM), jnp.float32),
            pltpu.VMEM((_EPW2,), jnp.int32),
            pltpu.VMEM((_EPW2,), jnp.int32),
            pltpu.VMEM((_EPW2,), jnp.int32),
            pltpu.VMEM((1, _EC2), jnp.int32),
            pltpu.VMEM((_EC2,), jnp.int32),
            pltpu.VMEM((_EC2,), jnp.int32),
            pltpu.VMEM((_EC2, DIM), jnp.float32),
            pltpu.VMEM((_EC2,), jnp.float32),
            pltpu.VMEM((_EC2,), jnp.float32),
            pltpu.VMEM((_EC2,), jnp.float32),
            pltpu.VMEM((16, DIM), jnp.float32),
            pltpu.SemaphoreType.DMA,
            pltpu.SemaphoreType.DMA,
        ],
    )
    def k(xw_hbm, src_hbm, dst_hbm, type_hbm, c0_hbm, c1_hbm, out_hbm,
          acc_sp, src_v, dst_v, type_v, didx2, gidx_c, cidx_c,
          rows_v, c0_v, c1_v, w_v, zbuf, sem, sem2):
        c = lax.axis_index("c")
        s = lax.axis_index("s")
        ebase = s * _EPW2
        nbase = c * _HALF

        for j in range(16):
            for i in range(DIM // SC_L):
                zbuf[j, pl.ds(i * SC_L, SC_L)] = jnp.zeros((SC_L,),
                                                           jnp.float32)
        wbase = s * (_ACC_ROWS // SC_NS)

        def _z(t, _):
            pltpu.sync_copy(zbuf, acc_sp.at[pl.ds(wbase + t * 16, 16)])
            return 0
        lax.fori_loop(0, 20, _z, 0)

        def _ld(t, _):
            sl_h = pl.ds(ebase + t * 1024, 1024)
            sl_v = pl.ds(t * 1024, 1024)
            pltpu.sync_copy(src_hbm.at[sl_h], src_v.at[sl_v])
            pltpu.sync_copy(dst_hbm.at[sl_h], dst_v.at[sl_v])
            pltpu.sync_copy(type_hbm.at[sl_h], type_v.at[sl_v])
            return 0
        lax.fori_loop(0, _EPW2 // 1024, _ld, 0)

        plsc.subcore_barrier()

        def _chunk(j, _):
            for i in range(_EC2 // SC_L):
                sl = pl.ds(j * _EC2 + i * SC_L, SC_L)
                co = pl.ds(i * SC_L, SC_L)
                t16 = type_v[sl]
                d16 = dst_v[sl]
                gidx_c[co] = t16 * N_NODES + src_v[sl]
                cidx_c[co] = (jnp.minimum(d16, N_NODES - 1) * NUM_RELATIONS
                              + t16)
                loc = d16 - nbase
                didx2[0, co] = jnp.where(
                    (loc >= 0) & (loc < _HALF), loc, _HALF)
            cp1 = pltpu.async_copy(xw_hbm.at[gidx_c], rows_v, sem)
            cp2 = pltpu.async_copy(c0_hbm.at[cidx_c], c0_v, sem2)
            cp3 = pltpu.async_copy(c1_hbm.at[cidx_c], c1_v, sem2)
            cp1.wait()
            cp2.wait()
            cp3.wait()
            for i in range(_EC2 // SC_L):
                co = pl.ds(i * SC_L, SC_L)
                w_v[co] = 1.0 / jnp.maximum(c0_v[co] + c1_v[co], 1.0)
            for g in range(_EC2 // SC_L):
                w16 = w_v[pl.ds(g * SC_L, SC_L)]
                for el in range(SC_L):
                    e = g * SC_L + el
                    w_s = w16[el]
                    for i in range(DIM // SC_L):
                        co = pl.ds(i * SC_L, SC_L)
                        rows_v[e, co] = rows_v[e, co] * w_s
            pltpu.sync_copy(rows_v, acc_sp.at[didx2.at[0]], add=True)
            return 0
        lax.fori_loop(0, _ECH2, _chunk, 0)

        plsc.subcore_barrier()

        def _wo(t, _):
            pltpu.sync_copy(acc_sp.at[pl.ds(wbase + t * 16, 16)], zbuf)
            pltpu.sync_copy(zbuf, out_hbm.at[c, pl.ds(wbase + t * 16, 16)])
            return 0
        lax.fori_loop(0, 20, _wo, 0)

    aggP = k(xwf, src, dst, edge_type, cnt0, cnt1)
    return jnp.concatenate([aggP[0, :_HALF], aggP[1, :_HALF]], axis=0)[None]


def _tf_aggregate(q, k, v, er, src, dst, edge_type):
    qd = q[dst].reshape(-1, HEADS, HEAD_DIM)
    e = er[edge_type]
    ke = (k[src] + e).reshape(-1, HEADS, HEAD_DIM)
    ve = (v[src] + e).reshape(-1, HEADS, HEAD_DIM)
    alpha = jnp.sum(qd * ke, axis=-1) * (1.0 / jnp.sqrt(float(HEAD_DIM)))
    ex = jnp.exp(alpha)
    den = jax.ops.segment_sum(ex, dst, num_segments=N_NODES)
    attn = ex / (den[dst] + 1e-16)
    return jax.ops.segment_sum(ve * attn[:, :, None], dst,
                               num_segments=N_NODES).reshape(N_NODES, DIM)


def kernel(params, node_ids, edge_index, edge_type):
    src, dst = edge_index[0], edge_index[1]
    npad = _EPAD - N_EDGES
    srcp = jnp.concatenate([src, jnp.zeros((npad,), jnp.int32)])
    dstp = jnp.concatenate([dst, jnp.full((npad,), N_NODES, jnp.int32)])
    typep = jnp.concatenate([edge_type, jnp.zeros((npad,), jnp.int32)])

    x = _entity_gather(params['entity_table'], node_ids)

    cntP = _rel_count(src, dst, edge_type)
    cnt0, cnt1 = cntP[0], cntP[1]

    for l in range(2):
        p = params['rgcn'][l]
        w = _small_matmul(p['comp'],
                          p['bases'].reshape(NUM_BASES, DIM * DIM))
        w = w.reshape(NUM_RELATIONS, DIM, DIM)
        xw = _xw_all_relations(x, w)
        aggP = _rgcn_aggregate(xw.reshape(NUM_RELATIONS * N_NODES, DIM),
                               srcp, dstp, typep, cnt0, cnt1)
        x = _combine(aggP, x, p['root'], p['bias'], p['ln_g'], p['ln_b'])

    mean = None
    for l in range(2):
        p = params['tf'][l]
        er = _small_matmul(params['relation_table'], p['We'])
        wqkv = jnp.concatenate([p['Wq'], p['Wk'], p['Wv']], axis=1)
        bqkv = jnp.concatenate([p['bq'], p['bk'], p['bv']], axis=0)
        qkv = _qkv(x, wqkv, bqkv)
        q, k, v = qkv[:, :DIM], qkv[:, DIM:2 * DIM], qkv[:, 2 * DIM:]
        agg = _tf_aggregate(q, k, v, er, src, dst, edge_type)[None]
        last = (l == 1)
        res = _combine(agg, x, p['Wskip'], p['bskip'], p['ln_g'], p['ln_b'],
                       residual=True, mean_out=last)
        x = res[0] if last else res
    graph_embedding = res[1]
    return x, graph_embedding


# speedup vs baseline: 1.1277x; 1.0004x over previous
import functools

import jax
import jax.numpy as jnp
from jax import lax
from jax.experimental import pallas as pl
from jax.experimental.pallas import tpu as pltpu
from jax.experimental.pallas import tpu_sc as plsc

_SC = plsc.get_sparse_core_info()
SC_NC, SC_NS, SC_L = _SC.num_cores, _SC.num_subcores, _SC.num_lanes
NW = SC_NC * SC_NS

N_NODES = 10000
N_EDGES = 320000
NUM_RELATIONS = 64
NUM_BASES = 30
DIM = 128
HEADS = 8
HEAD_DIM = DIM // HEADS

NT = 2000


def _matmul_kernel(a_ref, b_ref, o_ref):
    o_ref[...] = jnp.dot(a_ref[...], b_ref[...],
                         preferred_element_type=jnp.float32)


def _small_matmul(a, b):
    m, k = a.shape
    k2, n = b.shape
    return pl.pallas_call(
        _matmul_kernel,
        out_shape=jax.ShapeDtypeStruct((m, n), jnp.float32),
    )(a, b)


def _xw_kernel(x_ref, w_ref, o_ref):
    o_ref[0] = jnp.dot(x_ref[...], w_ref[0],
                       preferred_element_type=jnp.float32)


def _xw_all_relations(x, w):
    grid = (N_NODES // NT, NUM_RELATIONS)
    return pl.pallas_call(
        _xw_kernel,
        grid=grid,
        in_specs=[
            pl.BlockSpec((NT, DIM), lambda i, r: (i, 0)),
            pl.BlockSpec((1, DIM, DIM), lambda i, r: (r, 0, 0)),
        ],
        out_specs=pl.BlockSpec((1, NT, DIM), lambda i, r: (r, i, 0)),
        out_shape=jax.ShapeDtypeStruct((NUM_RELATIONS, N_NODES, DIM),
                                       jnp.float32),
    )(x, w)


def _combine_kernel(agg_ref, x_ref, w_ref, b_ref, g_ref, bb_ref, *rest,
                    residual, mean_out, scaled):
    if scaled:
        sc_ref, o_ref, m_ref = rest
    else:
        o_ref, m_ref = rest
        sc_ref = None
    x = x_ref[...]
    agg = jnp.sum(agg_ref[...], axis=0)
    if scaled:
        agg = agg * sc_ref[...]
    h = agg + jnp.dot(x, w_ref[...], preferred_element_type=jnp.float32)
    h = h + b_ref[...]
    mu = jnp.mean(h, axis=-1, keepdims=True)
    var = jnp.mean((h - mu) ** 2, axis=-1, keepdims=True)
    h = (h - mu) / jnp.sqrt(var + 1e-5) * g_ref[...] + bb_ref[...]
    h = jnp.maximum(h, 0.0)
    if residual:
        h = x + h
    o_ref[...] = h
    if mean_out:
        i = pl.program_id(0)

        @pl.when(i == 0)
        def _():
            m_ref[...] = jnp.zeros_like(m_ref)

        m_ref[...] += jnp.sum(h, axis=0, keepdims=True) * (1.0 / N_NODES)


def _combine(agg, x, w, b, g, bb, residual=False, mean_out=False,
             scale=None):
    nparts = agg.shape[0]
    grid = (N_NODES // NT,)
    kern = functools.partial(_combine_kernel, residual=residual,
                             mean_out=mean_out, scaled=scale is not None)
    out_shape = [jax.ShapeDtypeStruct((N_NODES, DIM), jnp.float32),
                 jax.ShapeDtypeStruct((1, DIM), jnp.float32)]
    in_specs = [
        pl.BlockSpec((nparts, NT, DIM), lambda i: (0, i, 0)),
        pl.BlockSpec((NT, DIM), lambda i: (i, 0)),
        pl.BlockSpec((DIM, DIM), lambda i: (0, 0)),
        pl.BlockSpec((DIM,), lambda i: (0,)),
        pl.BlockSpec((DIM,), lambda i: (0,)),
        pl.BlockSpec((DIM,), lambda i: (0,)),
    ]
    args = [agg, x, w, b, g, bb]
    if scale is not None:
        in_specs.append(pl.BlockSpec((NT, DIM), lambda i: (i, 0)))
        args.append(scale)
    outs = pl.pallas_call(
        kern,
        grid=grid,
        in_specs=in_specs,
        out_specs=[
            pl.BlockSpec((NT, DIM), lambda i: (i, 0)),
            pl.BlockSpec((1, DIM), lambda i: (0, 0)),
        ],
        out_shape=out_shape,
    )(*args)
    return outs if mean_out else outs[0]


def _rep_kernel(d_ref, o_ref):
    rec = 1.0 / (d_ref[0] + d_ref[1] + 1e-16)
    rep = jnp.broadcast_to(rec[:, :HEADS, None],
                           (rec.shape[0], HEADS, HEAD_DIM))
    o_ref[...] = rep.reshape(rec.shape[0], DIM)


def _den_rep(den):
    return pl.pallas_call(
        _rep_kernel,
        grid=(N_NODES // NT,),
        in_specs=[pl.BlockSpec((2, NT, 16), lambda i: (0, i, 0))],
        out_specs=pl.BlockSpec((NT, DIM), lambda i: (i, 0)),
        out_shape=jax.ShapeDtypeStruct((N_NODES, DIM), jnp.float32),
    )(den)


def _qkv_kernel(x_ref, w_ref, b_ref, o_ref):
    o_ref[...] = jnp.dot(x_ref[...], w_ref[...],
                         preferred_element_type=jnp.float32) + b_ref[...]


def _qkv(x, wqkv, bqkv):
    grid = (N_NODES // NT,)
    return pl.pallas_call(
        _qkv_kernel,
        grid=grid,
        in_specs=[
            pl.BlockSpec((NT, DIM), lambda i: (i, 0)),
            pl.BlockSpec((DIM, 3 * DIM), lambda i: (0, 0)),
            pl.BlockSpec((3 * DIM,), lambda i: (0,)),
        ],
        out_specs=pl.BlockSpec((NT, 3 * DIM), lambda i: (i, 0)),
        out_shape=jax.ShapeDtypeStruct((N_NODES, 3 * DIM), jnp.float32),
    )(x, wqkv, bqkv)


_GB = 10240
_G_PER_W = _GB // NW
_GC = 64


def _entity_gather(table, node_ids):
    ids = jnp.concatenate(
        [node_ids, jnp.zeros((_GB - N_NODES,), jnp.int32)])
    mesh = plsc.VectorSubcoreMesh(core_axis_name="c", subcore_axis_name="s")

    @functools.partial(
        pl.kernel, mesh=mesh,
        out_type=jax.ShapeDtypeStruct((_GB, DIM), jnp.float32),
        scratch_types=[
            pltpu.VMEM((_G_PER_W // _GC, _GC), jnp.int32),
            pltpu.VMEM((_GC, DIM), jnp.float32),
            pltpu.SemaphoreType.DMA,
        ],
    )
    def k(table_hbm, idx_hbm, out_hbm, idx_v, rows_v, sem):
        wid = lax.axis_index("s") * SC_NC + lax.axis_index("c")
        base = wid * _G_PER_W
        for j in range(_G_PER_W // _GC):
            pltpu.sync_copy(idx_hbm.at[pl.ds(base + j * _GC, _GC)],
                            idx_v.at[j])
            pltpu.async_copy(table_hbm.at[idx_v.at[j]], rows_v, sem).wait()
            pltpu.sync_copy(rows_v, out_hbm.at[pl.ds(base + j * _GC, _GC)])

    return k(table, ids)[:N_NODES]


_EPW = N_EDGES // NW
_EC = 80
_ECH = _EPW // _EC
_NSEG = N_NODES * NUM_RELATIONS
_SEG_PER_T = _NSEG // (NW // 2)
_ROW_PER_T = N_NODES // (NW // 2)


def _sc_mesh():
    return plsc.VectorSubcoreMesh(core_axis_name="c", subcore_axis_name="s")


def _iota16():
    return lax.iota(jnp.int32, SC_L)


def _permute16(x, idx):
    dnums = lax.GatherDimensionNumbers(
        offset_dims=(), collapsed_slice_dims=(0,), start_index_map=(0,))
    return lax.gather(x, idx[:, None], dnums, slice_sizes=(1,),
                      mode=lax.GatherScatterMode.PROMISE_IN_BOUNDS)


def _rel_count(src, dst, edge_type):
    @functools.partial(
        pl.kernel, mesh=_sc_mesh(),
        out_type=jax.ShapeDtypeStruct((2 * _NSEG,), jnp.float32),
        scratch_types=[
            pltpu.VMEM_SHARED((_NSEG,), jnp.float32),
            pltpu.VMEM((_EPW,), jnp.int32),
            pltpu.VMEM((_EPW,), jnp.int32),
            pltpu.VMEM((_ECH, _EC), jnp.int32),
            pltpu.VMEM((_EC,), jnp.float32),
            pltpu.VMEM((2000,), jnp.float32),
        ],
    )
    def k(dst_hbm, type_hbm, out_hbm, cnt_sp, dst_v, type_v, idx2, ones_v,
          zbuf):
        c = lax.axis_index("c")
        s = lax.axis_index("s")
        wid = s * SC_NC + c
        ebase = wid * _EPW

        def _zfill(i, _):
            zbuf[pl.ds(i * SC_L, SC_L)] = jnp.zeros((SC_L,), jnp.float32)
            return 0
        lax.fori_loop(0, 2000 // SC_L, _zfill, 0)
        for t in range(_SEG_PER_T // 2000):
            pltpu.sync_copy(zbuf, cnt_sp.at[pl.ds(s * _SEG_PER_T + t * 2000,
                                                  2000)])

        pltpu.sync_copy(dst_hbm.at[pl.ds(ebase, _EPW)], dst_v)
        pltpu.sync_copy(type_hbm.at[pl.ds(ebase, _EPW)], type_v)
        for i in range(_EC // SC_L):
            ones_v[pl.ds(i * SC_L, SC_L)] = jnp.ones((SC_L,), jnp.float32)

        def _mkidx(j, _):
            for i in range(_EC // SC_L):
                sl = pl.ds(j * _EC + i * SC_L, SC_L)
                idx2[j, pl.ds(i * SC_L, SC_L)] = (
                    dst_v[sl] * NUM_RELATIONS + type_v[sl])
            return 0
        lax.fori_loop(0, _ECH, _mkidx, 0)

        plsc.subcore_barrier()

        def _scat(j, _):
            pltpu.sync_copy(ones_v, cnt_sp.at[idx2.at[j]], add=True)
            return 0
        lax.fori_loop(0, _ECH, _scat, 0)

        plsc.subcore_barrier()
        for t in range(_SEG_PER_T // 2000):
            pltpu.sync_copy(cnt_sp.at[pl.ds(s * _SEG_PER_T + t * 2000, 2000)],
                            zbuf)
            pltpu.sync_copy(
                zbuf,
                out_hbm.at[pl.ds(c * _NSEG + s * _SEG_PER_T + t * 2000,
                                 2000)])

    return k(dst, edge_type).reshape(2, _NSEG)


_EPAD = 327680
_EPW2 = _EPAD // SC_NS
_EC2 = 64
_ECH2 = _EPW2 // _EC2
_HALF = N_NODES // 2
_ACC_ROWS = 5120


def _rgcn_aggregate(xwf, src, dst, edge_type, cnt0, cnt1):
    @functools.partial(
        pl.kernel, mesh=_sc_mesh(),
        out_type=jax.ShapeDtypeStruct((2, _ACC_ROWS, DIM), jnp.float32),
        scratch_types=[
            pltpu.VMEM_SHARED((_ACC_ROWS, DIM), jnp.float32),
            pltpu.VMEM((_EPW2,), jnp.int32),
            pltpu.VMEM((_EPW2,), jnp.int32),
            pltpu.VMEM((_EPW2,), jnp.int32),
            pltpu.VMEM((1, _EC2), jnp.int32),
            pltpu.VMEM((_EC2,), jnp.int32),
            pltpu.VMEM((_EC2,), jnp.int32),
            pltpu.VMEM((_EC2, DIM), jnp.float32),
            pltpu.VMEM((_EC2,), jnp.float32),
            pltpu.VMEM((_EC2,), jnp.float32),
            pltpu.VMEM((_EC2,), jnp.float32),
            pltpu.VMEM((16, DIM), jnp.float32),
            pltpu.SemaphoreType.DMA,
            pltpu.SemaphoreType.DMA,
        ],
    )
    def k(xw_hbm, src_hbm, dst_hbm, type_hbm, c0_hbm, c1_hbm, out_hbm,
          acc_sp, src_v, dst_v, type_v, didx2, gidx_c, cidx_c,
          rows_v, c0_v, c1_v, w_v, zbuf, sem, sem2):
        c = lax.axis_index("c")
        s = lax.axis_index("s")
        ebase = s * _EPW2
        nbase = c * _HALF

        for j in range(16):
            for i in range(DIM // SC_L):
                zbuf[j, pl.ds(i * SC_L, SC_L)] = jnp.zeros((SC_L,),
                                                           jnp.float32)
        wbase = s * (_ACC_ROWS // SC_NS)

        def _z(t, _):
            pltpu.sync_copy(zbuf, acc_sp.at[pl.ds(wbase + t * 16, 16)])
            return 0
        lax.fori_loop(0, 20, _z, 0)

        def _ld(t, _):
            sl_h = pl.ds(ebase + t * 1024, 1024)
            sl_v = pl.ds(t * 1024, 1024)
            pltpu.sync_copy(src_hbm.at[sl_h], src_v.at[sl_v])
            pltpu.sync_copy(dst_hbm.at[sl_h], dst_v.at[sl_v])
            pltpu.sync_copy(type_hbm.at[sl_h], type_v.at[sl_v])
            return 0
        lax.fori_loop(0, _EPW2 // 1024, _ld, 0)

        plsc.subcore_barrier()

        def _chunk(j, _):
            for i in range(_EC2 // SC_L):
                sl = pl.ds(j * _EC2 + i * SC_L, SC_L)
                co = pl.ds(i * SC_L, SC_L)
                t16 = type_v[sl]
                d16 = dst_v[sl]
                gidx_c[co] = t16 * N_NODES + src_v[sl]
                cidx_c[co] = (jnp.minimum(d16, N_NODES - 1) * NUM_RELATIONS
                              + t16)
                loc = d16 - nbase
                didx2[0, co] = jnp.where(
                    (loc >= 0) & (loc < _HALF), loc, _HALF)
            cp1 = pltpu.async_copy(xw_hbm.at[gidx_c], rows_v, sem)
            cp2 = pltpu.async_copy(c0_hbm.at[cidx_c], c0_v, sem2)
            cp3 = pltpu.async_copy(c1_hbm.at[cidx_c], c1_v, sem2)
            cp1.wait()
            cp2.wait()
            cp3.wait()
            for i in range(_EC2 // SC_L):
                co = pl.ds(i * SC_L, SC_L)
                w_v[co] = 1.0 / jnp.maximum(c0_v[co] + c1_v[co], 1.0)
            for g in range(_EC2 // SC_L):
                w16 = w_v[pl.ds(g * SC_L, SC_L)]
                for el in range(SC_L):
                    e = g * SC_L + el
                    w_s = w16[el]
                    for i in range(DIM // SC_L):
                        co = pl.ds(i * SC_L, SC_L)
                        rows_v[e, co] = rows_v[e, co] * w_s
            pltpu.sync_copy(rows_v, acc_sp.at[didx2.at[0]], add=True)
            return 0
        lax.fori_loop(0, _ECH2, _chunk, 0)

        plsc.subcore_barrier()

        def _wo(t, _):
            pltpu.sync_copy(acc_sp.at[pl.ds(wbase + t * 16, 16)], zbuf)
            pltpu.sync_copy(zbuf, out_hbm.at[c, pl.ds(wbase + t * 16, 16)])
            return 0
        lax.fori_loop(0, 20, _wo, 0)

    aggP = k(xwf, src, dst, edge_type, cnt0, cnt1)
    return jnp.concatenate([aggP[0, :_HALF], aggP[1, :_HALF]], axis=0)[None]


_TPW = _EPAD // NW
_T1C = 32
_TCH = _TPW // _T1C
_DEN_ROWS = 10112


def _tf_pass1(q, k, er, src, dst, edge_type):
    @functools.partial(
        pl.kernel, mesh=_sc_mesh(),
        out_type=[jax.ShapeDtypeStruct((_EPAD * 16,), jnp.float32),
                  jax.ShapeDtypeStruct((2 * _DEN_ROWS * 16,), jnp.float32)],
        scratch_types=[
            pltpu.VMEM_SHARED((_DEN_ROWS, 16), jnp.float32),
            pltpu.VMEM((_TPW,), jnp.int32),
            pltpu.VMEM((_TPW,), jnp.int32),
            pltpu.VMEM((_TPW,), jnp.int32),
            pltpu.VMEM((DIM * NUM_RELATIONS,), jnp.float32),
            pltpu.VMEM((_T1C,), jnp.int32),
            pltpu.VMEM((_T1C,), jnp.int32),
            pltpu.VMEM((1, _T1C), jnp.int32),
            pltpu.VMEM((_T1C, DIM), jnp.float32),
            pltpu.VMEM((_T1C, DIM), jnp.float32),
            pltpu.VMEM((_T1C, 16), jnp.float32),
            pltpu.VMEM((_T1C * 16,), jnp.float32),
            pltpu.VMEM((8, 16), jnp.float32),
            pltpu.VMEM((DIM,), jnp.float32),
            pltpu.SemaphoreType.DMA,
            pltpu.SemaphoreType.DMA,
        ],
    )
    def k1(q_hbm, k_hbm, er_hbm, src_hbm, dst_hbm, type_hbm,
           ex_hbm, den_hbm, den_sp, src_v, dst_v, type_v, er_f,
           qidx, kidx, didx_r, qrows, krows, exd, exf, zbuf, zflat,
           sem, sem2):
        c = lax.axis_index("c")
        s = lax.axis_index("s")
        wid = s * SC_NC + c
        ebase = wid * _TPW

        for j in range(8):
            zbuf[j, pl.ds(0, SC_L)] = jnp.zeros((SC_L,), jnp.float32)
        dbase = s * (_DEN_ROWS // SC_NS)

        def _z(t, _):
            pltpu.sync_copy(zbuf, den_sp.at[pl.ds(dbase + t * 8, 8)])
            return 0
        lax.fori_loop(0, 79, _z, 0)

        def _lde(t, _):
            sl = pl.ds(t * 1024, 1024)
            pltpu.sync_copy(er_hbm.at[sl], er_f.at[sl])
            return 0
        lax.fori_loop(0, DIM * NUM_RELATIONS // 1024, _lde, 0)

        def _ld(t, _):
            sl_h = pl.ds(ebase + t * 256, 256)
            sl_v = pl.ds(t * 256, 256)
            pltpu.sync_copy(src_hbm.at[sl_h], src_v.at[sl_v])
            pltpu.sync_copy(dst_hbm.at[sl_h], dst_v.at[sl_v])
            pltpu.sync_copy(type_hbm.at[sl_h], type_v.at[sl_v])
            return 0
        lax.fori_loop(0, _TPW // 256, _ld, 0)

        plsc.subcore_barrier()

        def _chunk(j, _):
            for i in range(_T1C // SC_L):
                sl = pl.ds(j * _T1C + i * SC_L, SC_L)
                co = pl.ds(i * SC_L, SC_L)
                d16 = dst_v[sl]
                qidx[co] = jnp.minimum(d16, N_NODES - 1)
                kidx[co] = jnp.minimum(src_v[sl], N_NODES - 1)
                didx_r[0, co] = jnp.minimum(d16, N_NODES)
            cp1 = pltpu.async_copy(q_hbm.at[qidx], qrows, sem)
            cp2 = pltpu.async_copy(k_hbm.at[kidx], krows, sem2)
            cp1.wait()
            cp2.wait()
            io = _iota16()
            perms = [io ^ kk for kk in (8, 4, 2, 1)]
            for g in range(_T1C // SC_L):
                t16 = type_v[pl.ds(j * _T1C + g * SC_L, SC_L)]
                for el in range(SC_L):
                    e = g * SC_L + el
                    t_s = t16[el]
                    acc = jnp.zeros((SC_L,), jnp.float32)
                    for h in range(HEADS):
                        co = pl.ds(h * HEAD_DIM, HEAD_DIM)
                        keh = (krows[e, co]
                               + er_f[pl.ds(t_s * DIM + h * HEAD_DIM,
                                            HEAD_DIM)])
                        ss = qrows[e, co] * keh
                        for p in perms:
                            ss = ss + _permute16(ss, p)
                        acc = acc + jnp.where(io == h, ss, 0.0)
                    ex16 = jnp.exp(acc * 0.25)
                    exd[e, pl.ds(0, SC_L)] = ex16
                    exf[pl.ds(e * SC_L, SC_L)] = ex16
            pltpu.sync_copy(exd, den_sp.at[didx_r.at[0]], add=True)
            pltpu.sync_copy(
                exf, ex_hbm.at[pl.ds((ebase + j * _T1C) * 16, _T1C * 16)])
            return 0
        lax.fori_loop(0, _TCH, _chunk, 0)

        plsc.subcore_barrier()

        def _wo(t, _):
            pltpu.sync_copy(den_sp.at[pl.ds(dbase + t * 8, 8)], zbuf)
            for j in range(8):
                zflat[pl.ds(j * SC_L, SC_L)] = zbuf[j, pl.ds(0, SC_L)]
            pltpu.sync_copy(
                zflat,
                den_hbm.at[pl.ds((c * _DEN_ROWS + dbase + t * 8) * 16,
                                 DIM)])
            return 0
        lax.fori_loop(0, 79, _wo, 0)

    return k1(q, k, er, src, dst, edge_type)


def _tf_pass2(v, er, src, dst, edge_type, ex):
    @functools.partial(
        pl.kernel, mesh=_sc_mesh(),
        out_type=jax.ShapeDtypeStruct((2, _ACC_ROWS, DIM), jnp.float32),
        scratch_types=[
            pltpu.VMEM_SHARED((_ACC_ROWS, DIM), jnp.float32),
            pltpu.VMEM((_EPW2,), jnp.int32),
            pltpu.VMEM((_EPW2,), jnp.int32),
            pltpu.VMEM((_EPW2,), jnp.int32),
            pltpu.VMEM((DIM * NUM_RELATIONS,), jnp.float32),
            pltpu.VMEM((_EC2,), jnp.int32),
            pltpu.VMEM((1, _EC2), jnp.int32),
            pltpu.VMEM((_EC2, DIM), jnp.float32),
            pltpu.VMEM((_EC2 * 16,), jnp.float32),
            pltpu.VMEM((16, DIM), jnp.float32),
            pltpu.SemaphoreType.DMA,
        ],
    )
    def k2(v_hbm, er_hbm, src_hbm, dst_hbm, type_hbm, ex_hbm,
           out_hbm, acc_sp, src_v, dst_v, type_v, er_f,
           vidx, didx_r, vrows, exf, zbuf, sem):
        c = lax.axis_index("c")
        s = lax.axis_index("s")
        ebase = s * _EPW2
        nbase = c * _HALF

        for j in range(16):
            for i in range(DIM // SC_L):
                zbuf[j, pl.ds(i * SC_L, SC_L)] = jnp.zeros((SC_L,),
                                                           jnp.float32)
        wbase = s * (_ACC_ROWS // SC_NS)

        def _z(t, _):
            pltpu.sync_copy(zbuf, acc_sp.at[pl.ds(wbase + t * 16, 16)])
            return 0
        lax.fori_loop(0, 20, _z, 0)

        def _lde(t, _):
            sl = pl.ds(t * 1024, 1024)
            pltpu.sync_copy(er_hbm.at[sl], er_f.at[sl])
            return 0
        lax.fori_loop(0, DIM * NUM_RELATIONS // 1024, _lde, 0)

        def _ld(t, _):
            sl_h = pl.ds(ebase + t * 1024, 1024)
            sl_v = pl.ds(t * 1024, 1024)
            pltpu.sync_copy(src_hbm.at[sl_h], src_v.at[sl_v])
            pltpu.sync_copy(dst_hbm.at[sl_h], dst_v.at[sl_v])
            pltpu.sync_copy(type_hbm.at[sl_h], type_v.at[sl_v])
            return 0
        lax.fori_loop(0, _EPW2 // 1024, _ld, 0)

        plsc.subcore_barrier()

        def _chunk(j, _):
            for i in range(_EC2 // SC_L):
                sl = pl.ds(j * _EC2 + i * SC_L, SC_L)
                co = pl.ds(i * SC_L, SC_L)
                d16 = dst_v[sl]
                vidx[co] = jnp.minimum(src_v[sl], N_NODES - 1)
                loc = d16 - nbase
                didx_r[0, co] = jnp.where(
                    (loc >= 0) & (loc < _HALF), loc, _HALF)
            cp1 = pltpu.async_copy(v_hbm.at[vidx], vrows, sem)
            pltpu.sync_copy(
                ex_hbm.at[pl.ds((ebase + j * _EC2) * 16, _EC2 * 16)], exf)
            cp1.wait()
            for g in range(_EC2 // SC_L):
                t16 = type_v[pl.ds(j * _EC2 + g * SC_L, SC_L)]
                for el in range(SC_L):
                    e = g * SC_L + el
                    t_s = t16[el]
                    a16 = exf[pl.ds(e * 16, SC_L)]
                    for h in range(HEADS):
                        co = pl.ds(h * HEAD_DIM, HEAD_DIM)
                        veh = (vrows[e, co]
                               + er_f[pl.ds(t_s * DIM + h * HEAD_DIM,
                                            HEAD_DIM)])
                        vrows[e, co] = veh * a16[h]
            pltpu.sync_copy(vrows, acc_sp.at[didx_r.at[0]], add=True)
            return 0
        lax.fori_loop(0, _ECH2, _chunk, 0)

        plsc.subcore_barrier()

        def _wo(t, _):
            pltpu.sync_copy(acc_sp.at[pl.ds(wbase + t * 16, 16)], zbuf)
            pltpu.sync_copy(zbuf, out_hbm.at[c, pl.ds(wbase + t * 16, 16)])
            return 0
        lax.fori_loop(0, 20, _wo, 0)

    aggP = k2(v, er, src, dst, edge_type, ex)
    return jnp.concatenate([aggP[0, :_HALF], aggP[1, :_HALF]], axis=0)[None]


def kernel(params, node_ids, edge_index, edge_type):
    src, dst = edge_index[0], edge_index[1]
    npad = _EPAD - N_EDGES
    srcp = jnp.concatenate([src, jnp.zeros((npad,), jnp.int32)])
    dstp = jnp.concatenate([dst, jnp.full((npad,), N_NODES, jnp.int32)])
    typep = jnp.concatenate([edge_type, jnp.zeros((npad,), jnp.int32)])

    x = _entity_gather(params['entity_table'], node_ids)

    cntP = _rel_count(src, dst, edge_type)
    cnt0, cnt1 = cntP[0], cntP[1]

    for l in range(2):
        p = params['rgcn'][l]
        w = _small_matmul(p['comp'],
                          p['bases'].reshape(NUM_BASES, DIM * DIM))
        w = w.reshape(NUM_RELATIONS, DIM, DIM)
        xw = _xw_all_relations(x, w)
        aggP = _rgcn_aggregate(xw.reshape(NUM_RELATIONS * N_NODES, DIM),
                               srcp, dstp, typep, cnt0, cnt1)
        x = _combine(aggP, x, p['root'], p['bias'], p['ln_g'], p['ln_b'])

    mean = None
    for l in range(2):
        p = params['tf'][l]
        er = _small_matmul(params['relation_table'], p['We'])
        wqkv = jnp.concatenate([p['Wq'], p['Wk'], p['Wv']], axis=1)
        bqkv = jnp.concatenate([p['bq'], p['bk'], p['bv']], axis=0)
        qkv = _qkv(x, wqkv, bqkv)
        q, k, v = qkv[:, :DIM], qkv[:, DIM:2 * DIM], qkv[:, 2 * DIM:]
        erf = er.reshape(-1)
        ex, den = _tf_pass1(q, k, erf, srcp, dstp, typep)
        den = den.reshape(2, _DEN_ROWS, 16)[:, :N_NODES]
        exr = ex.reshape(_EPAD, 16)[:N_EDGES, :HEADS]
        denr = jax.ops.segment_sum(exr, dst, num_segments=N_NODES)
        attn = exr / (denr[dst] + 1e-16)
        ve = (v[src] + er[edge_type]).reshape(-1, HEADS, HEAD_DIM)
        agg = jax.ops.segment_sum(ve * attn[:, :, None], dst,
                                  num_segments=N_NODES)
        agg = agg.reshape(N_NODES, DIM)[None]
        last = (l == 1)
        res = _combine(agg, x, p['Wskip'], p['bskip'], p['ln_g'], p['ln_b'],
                       residual=True, mean_out=last)
        x = res[0] if last else res
    graph_embedding = res[1]
    return x, graph_embedding
